# Initial kernel scaffold; baseline (speedup 1.0000x reference)
import jax
import jax.numpy as jnp
from jax.experimental import pallas as pl


def kernel(x, txt_emb, pos_emb):
    raise NotImplementedError("write your pallas kernel here")


# speedup vs baseline: 2.5636x; 2.5636x over previous
import functools

import jax
import jax.numpy as jnp
from jax import lax
from jax.experimental import pallas as pl
from jax.experimental.pallas import tpu as pltpu
from jax.experimental.pallas import tpu_sc as plsc

B, L, D = 16, 2048, 128
NC, NS = 2, 16
NW = NC * NS
PER_W = (B * L) // NW
CH = 128
NCH = PER_W // CH

_mesh = plsc.VectorSubcoreMesh(core_axis_name="c", subcore_axis_name="s")


@functools.partial(
    pl.kernel,
    out_type=jax.ShapeDtypeStruct((B * L, D), jnp.float32),
    mesh=_mesh,
    scratch_types=[
        pltpu.VMEM((NCH, CH), jnp.int32),
        pltpu.VMEM((CH, D), jnp.float32),
        pltpu.SemaphoreType.DMA,
    ],
)
def _tok_pos(x_hbm, txt_hbm, pos_hbm, out_hbm, idx_v, rows_v, gsem):
    wid = lax.axis_index("s") * NC + lax.axis_index("c")
    base = wid * PER_W
    pos_base = lax.rem(base, L)
    pltpu.sync_copy(x_hbm.at[wid], idx_v)
    for j in range(NCH):
        pltpu.sync_copy(pos_hbm.at[pl.ds(pos_base + j * CH, CH)], rows_v)
        pltpu.async_copy(txt_hbm.at[idx_v.at[j]], rows_v, gsem, add=True).wait()
        pltpu.sync_copy(rows_v, out_hbm.at[pl.ds(base + j * CH, CH)])


def kernel(x, txt_emb, pos_emb):
    xf = x.reshape(NW, NCH, CH).astype(jnp.int32)
    out = _tok_pos(xf, txt_emb, pos_emb)
    return out.reshape(B, L, D)


# speedup vs baseline: 3.1085x; 1.2126x over previous
import functools

import jax
import jax.numpy as jnp
from jax import lax
from jax.experimental import pallas as pl
from jax.experimental.pallas import tpu as pltpu
from jax.experimental.pallas import tpu_sc as plsc

B, L, D = 16, 2048, 128
NC, NS = 2, 16
NW = NC * NS
PER_W = (B * L) // NW
CH = 128
NCH = PER_W // CH

NBUF = 4

_mesh = plsc.VectorSubcoreMesh(core_axis_name="c", subcore_axis_name="s")


@functools.partial(
    pl.kernel,
    out_type=jax.ShapeDtypeStruct((B * L, D), jnp.float32),
    mesh=_mesh,
    scratch_types=[
        pltpu.VMEM((NCH, CH), jnp.int32),
        pltpu.VMEM((NBUF, CH, D), jnp.float32),
        pltpu.SemaphoreType.DMA((NBUF,)),
        pltpu.SemaphoreType.DMA((NBUF,)),
        pltpu.SemaphoreType.DMA((NBUF,)),
    ],
)
def _tok_pos(x_hbm, txt_hbm, pos_hbm, out_hbm, idx_v, rows_v, psem, gsem, osem):
    wid = lax.axis_index("s") * NC + lax.axis_index("c")
    base = wid * PER_W
    pos_base = lax.rem(base, L)
    pltpu.sync_copy(x_hbm.at[wid], idx_v)

    def prefill(j):
        b = j % NBUF
        return pltpu.async_copy(
            pos_hbm.at[pl.ds(pos_base + j * CH, CH)], rows_v.at[b], psem.at[b])

    def gather(j):
        b = j % NBUF
        return pltpu.async_copy(
            txt_hbm.at[idx_v.at[j]], rows_v.at[b], gsem.at[b], add=True)

    def writeback(j):
        b = j % NBUF
        return pltpu.async_copy(
            rows_v.at[b], out_hbm.at[pl.ds(base + j * CH, CH)], osem.at[b])

    pre, gat, out = {}, {}, {}
    for j in range(NBUF):
        pre[j] = prefill(j)
    for j in range(NBUF):
        pre[j].wait()
        gat[j] = gather(j)
    for j in range(NBUF):
        gat[j].wait()
        out[j] = writeback(j)
    for j in range(NBUF, NCH):
        out[j - NBUF].wait()
        pre[j] = prefill(j)
        pre[j].wait()
        gat[j] = gather(j)
    for j in range(NBUF, NCH):
        gat[j].wait()
        out[j] = writeback(j)
    for j in range(NCH - NBUF, NCH):
        out[j].wait()


def kernel(x, txt_emb, pos_emb):
    xf = x.reshape(NW, NCH, CH).astype(jnp.int32)
    out = _tok_pos(xf, txt_emb, pos_emb)
    return out.reshape(B, L, D)


# speedup vs baseline: 3.7788x; 1.2156x over previous
import functools

import jax
import jax.numpy as jnp
from jax import lax
from jax.experimental import pallas as pl
from jax.experimental.pallas import tpu as pltpu
from jax.experimental.pallas import tpu_sc as plsc

B, L, D = 16, 2048, 128
NC, NS = 2, 16
NW = NC * NS
CW = L // NW
NCHUNK = B
LANES = 16
NBUF = 8

_mesh = plsc.VectorSubcoreMesh(core_axis_name="c", subcore_axis_name="s")


@functools.partial(
    pl.kernel,
    out_type=jax.ShapeDtypeStruct((B * L, D), jnp.float32),
    mesh=_mesh,
    scratch_types=[
        pltpu.VMEM((NCHUNK, CW), jnp.int32),
        pltpu.VMEM((CW, D), jnp.float32),
        pltpu.VMEM((NBUF, CW, D), jnp.float32),
        pltpu.SemaphoreType.DMA((NBUF,)),
        pltpu.SemaphoreType.DMA((NBUF,)),
    ],
)
def _tok_pos(x_hbm, txt_hbm, pos_hbm, out_hbm, idx_v, pos_v, rows_v, gsem, osem):
    wid = lax.axis_index("s") * NC + lax.axis_index("c")
    pltpu.sync_copy(x_hbm.at[wid], idx_v)
    pltpu.sync_copy(pos_hbm.at[pl.ds(wid * CW, CW)], pos_v)

    def gather(j):
        bb = j % NBUF
        return pltpu.async_copy(
            txt_hbm.at[idx_v.at[j]], rows_v.at[bb], gsem.at[bb])

    def writeback(j):
        bb = j % NBUF
        return pltpu.async_copy(
            rows_v.at[bb], out_hbm.at[pl.ds(j * L + wid * CW, CW)], osem.at[bb])

    def add_pos(bb):
        rv = rows_v.at[bb]

        def row_body(r, carry):
            for t in range(D // LANES):
                sl = pl.ds(t * LANES, LANES)
                rv[r, sl] = rv[r, sl] + pos_v[r, sl]
            return carry

        lax.fori_loop(0, CW, row_body, 0)

    gat, out = {}, {}
    for j in range(NBUF):
        gat[j] = gather(j)
    for j in range(NCHUNK):
        if j >= 1 and (j - 1) + NBUF < NCHUNK:
            out[j - 1].wait()
            gat[j - 1 + NBUF] = gather(j - 1 + NBUF)
        gat[j].wait()
        add_pos(j % NBUF)
        out[j] = writeback(j)
    for j in range(NCHUNK - NBUF, NCHUNK):
        out[j].wait()


def kernel(x, txt_emb, pos_emb):
    xr = x.reshape(B, NW, CW).swapaxes(0, 1).astype(jnp.int32)
    out = _tok_pos(xr, txt_emb, pos_emb)
    return out.reshape(B, L, D)
